# Initial kernel scaffold; baseline (speedup 1.0000x reference)
import jax
import jax.numpy as jnp
from jax.experimental import pallas as pl


def kernel(x, y, cluster_W, cluster_b, W, b):
    raise NotImplementedError("write your pallas kernel here")


# speedup vs baseline: 1.8265x; 1.8265x over previous
import jax
import jax.numpy as jnp
from jax.experimental import pallas as pl
from jax.experimental.pallas import tpu as pltpu

VOCAB = 100000
CUT1, CUT2, CUT3 = 2000, 10000, 50000
VT = 512
NT = (VOCAB + VT - 1) // VT
LPAD = 2048


def _adaptive_kernel(y_ref, x_ref, w_ref, b_ref, cw_ref, cb_ref,
                     out_ref, s_acc, t_acc):
    i = pl.program_id(0)

    @pl.when(i == 0)
    def _init():
        s_acc[:] = jnp.zeros_like(s_acc)
        t_acc[:] = jnp.zeros_like(t_acc)
        out_ref[:] = jnp.zeros_like(out_ref)

    logits = jnp.dot(x_ref[:], w_ref[:],
                     preferred_element_type=jnp.float32) + b_ref[:]
    col = i * VT + jax.lax.broadcasted_iota(jnp.int32, (1, VT), 1)
    col_cl = jnp.where(
        col < VOCAB,
        (col >= CUT1).astype(jnp.int32) + (col >= CUT2) + (col >= CUT3),
        -1)
    y = y_ref[:]
    tok_cl = ((y >= CUT1).astype(jnp.int32) + (y >= CUT2) + (y >= CUT3))
    mask = col_cl == tok_cl
    s_acc[:] += jnp.sum(jnp.where(mask, jnp.exp(logits), 0.0),
                        axis=1, keepdims=True)
    tmask = col == y
    t_acc[:] += jnp.sum(jnp.where(tmask, logits, 0.0),
                        axis=1, keepdims=True)

    @pl.when(i == NT - 1)
    def _finish():
        cl = jnp.dot(x_ref[:], cw_ref[:],
                     preferred_element_type=jnp.float32) + cb_ref[:]
        m = jnp.max(cl, axis=1, keepdims=True)
        lse_c = m + jnp.log(jnp.sum(jnp.exp(cl - m), axis=1, keepdims=True))
        ccol = jax.lax.broadcasted_iota(jnp.int32, (1, cl.shape[1]), 1)
        cll = jnp.sum(jnp.where(ccol == tok_cl, cl - lse_c, 0.0),
                      axis=1, keepdims=True)
        out_ref[:] = -(cll + t_acc[:] - jnp.log(s_acc[:]))


def kernel(x, y, cluster_W, cluster_b, W, b):
    x = x[:, :-1]
    bsz, l, h = x.shape
    xf = x.reshape(bsz * l, h)
    yf = y.reshape(-1)
    n = xf.shape[0]
    xp = jnp.pad(xf, ((0, LPAD - n), (0, 0)))
    yp = jnp.pad(yf, (0, LPAD - n), constant_values=-1).reshape(LPAD, 1)

    out = pl.pallas_call(
        _adaptive_kernel,
        grid=(NT,),
        in_specs=[
            pl.BlockSpec((LPAD, 1), lambda i: (0, 0)),
            pl.BlockSpec((LPAD, h), lambda i: (0, 0)),
            pl.BlockSpec((h, VT), lambda i: (0, i)),
            pl.BlockSpec((1, VT), lambda i: (0, i)),
            pl.BlockSpec(cluster_W.shape, lambda i: (0, 0)),
            pl.BlockSpec(cluster_b.shape, lambda i: (0, 0)),
        ],
        out_specs=pl.BlockSpec((LPAD, 1), lambda i: (0, 0)),
        out_shape=jax.ShapeDtypeStruct((LPAD, 1), jnp.float32),
        scratch_shapes=[
            pltpu.VMEM((LPAD, 1), jnp.float32),
            pltpu.VMEM((LPAD, 1), jnp.float32),
        ],
        compiler_params=pltpu.CompilerParams(
            dimension_semantics=("arbitrary",)),
    )(yp, xp, W, b, cluster_W, cluster_b)
    return out[:n, 0]
